# baseline (speedup 1.0000x reference)
import functools

import jax
import jax.numpy as jnp
import numpy as np
from jax import lax
from jax.experimental import pallas as pl
from jax.experimental.pallas import tpu as pltpu
from jax.experimental.pallas import tpu_sc as plsc

E = 320000
N = 10000
D = 128
H = 4
DK = 32

E_PAD = 327680
NPAD = 10112
CHUNK = 128
NROWS = E_PAD // CHUNK
NTILES = 32
RPT = NROWS // NTILES
RPT_B = NROWS // 16
EPT = E_PAD // NTILES
NPT = NPAD // 16

BE = 2560
NBLK = E // BE
NBLK_D = E_PAD // BE

_INV_SQRT_DK = 1.0 / np.sqrt(DK)
_CLAMP = 60.0

_mesh = plsc.VectorSubcoreMesh(core_axis_name="c", subcore_axis_name="s")


def _edge_proj_body(h_ref, q_ref, wk_ref, wv_ref, w1h_ref, w1q_ref, w2_ref,
                    b1_ref, b2_ref, rep_ref, o8_ref, out_ref):
    h = h_ref[...]
    q = q_ref[...]
    k = lax.dot_general(h, wk_ref[...], (((1,), (1,)), ((), ())),
                        preferred_element_type=jnp.float32)
    prod = q * k
    s_t = lax.dot_general(rep_ref[...], prod, (((1,), (1,)), ((), ())),
                          preferred_element_type=jnp.float32) * _INV_SQRT_DK
    ex_t = jnp.exp(jnp.minimum(s_t, _CLAMP))
    o8_ref[pl.ds(0, H), :] = ex_t
    o8_ref[pl.ds(H, 1), :] = jnp.sum(ex_t, axis=0, keepdims=True)
    hid = lax.dot_general(h, w1h_ref[...], (((1,), (1,)), ((), ())),
                          preferred_element_type=jnp.float32)
    hid = hid + lax.dot_general(q, w1q_ref[...], (((1,), (1,)), ((), ())),
                                preferred_element_type=jnp.float32)
    hid = jnp.maximum(hid + b1_ref[...], 0.0)
    wp_t = lax.dot_general(w2_ref[...], hid, (((1,), (1,)), ((), ())),
                           preferred_element_type=jnp.float32)
    o8_ref[pl.ds(H + 1, 1), :] = wp_t + b2_ref[0, 0]
    v = lax.dot_general(h, wv_ref[...], (((1,), (1,)), ((), ())),
                        preferred_element_type=jnp.float32)
    scale = lax.dot_general(ex_t, rep_ref[...], (((0,), (0,)), ((), ())),
                            preferred_element_type=jnp.float32)
    out_ref[...] = v * scale


def _edge_proj(h, q, wk, wv, w1h, w1q, w2, b1, b2, rep):
    full = lambda shp: pl.BlockSpec(shp, lambda i: (0, 0))
    clamped = lambda i: (jnp.minimum(i, NBLK - 1), 0)
    return pl.pallas_call(
        _edge_proj_body,
        grid=(NBLK_D,),
        in_specs=[
            pl.BlockSpec((BE, D), clamped),
            pl.BlockSpec((BE, D), clamped),
            full((D, D)), full((D, D)), full((D, D)), full((D, D)),
            full((1, D)), full((1, D)), full((1, 1)),
            full((H, D)),
        ],
        out_specs=[
            pl.BlockSpec((8, BE), lambda i: (0, i)),
            pl.BlockSpec((BE, D), lambda i: (i, 0)),
        ],
        out_shape=[
            jax.ShapeDtypeStruct((8, E_PAD), jnp.float32),
            jax.ShapeDtypeStruct((E_PAD, D), jnp.float32),
        ],
    )(h, q, wk, wv, w1h, w1q, w2, b1, b2, rep)


@functools.partial(
    pl.kernel,
    out_type=jax.ShapeDtypeStruct((NPAD,), jnp.float32),
    mesh=_mesh,
    scratch_types=[
        pltpu.VMEM_SHARED((NPAD,), jnp.float32),
        pltpu.VMEM((RPT_B, CHUNK), jnp.int32),
        pltpu.VMEM((RPT_B, CHUNK), jnp.float32),
    ],
)
def _seg_sum(sexp_hbm, idx_hbm, zeros_hbm, p_hbm, acc, idxbuf, updbuf):
    c = lax.axis_index("c")
    s = lax.axis_index("s")

    @pl.when(c == 0)
    def _():
        base = s * RPT_B

        @pl.when(s == 0)
        def _():
            pltpu.sync_copy(zeros_hbm, acc)

        pltpu.sync_copy(idx_hbm.at[pl.ds(base, RPT_B)], idxbuf)
        pltpu.sync_copy(sexp_hbm.at[pl.ds(base, RPT_B)], updbuf)
        plsc.subcore_barrier()

        def body(j, carry):
            pltpu.sync_copy(updbuf.at[j], acc.at[idxbuf.at[j]], add=True)
            return carry

        lax.fori_loop(0, RPT_B, body, 0)
        plsc.subcore_barrier()

        @pl.when(s == 0)
        def _():
            pltpu.sync_copy(acc, p_hbm)


@functools.partial(
    pl.kernel,
    out_type=jax.ShapeDtypeStruct((NROWS, CHUNK), jnp.float32),
    mesh=_mesh,
    scratch_types=[
        pltpu.VMEM_SHARED((NPAD,), jnp.float32),
        pltpu.VMEM((RPT, CHUNK), jnp.int32),
        pltpu.VMEM((RPT, CHUNK), jnp.float32),
        pltpu.SemaphoreType.DMA,
    ],
)
def _seg_gather(idx_hbm, p_hbm, d_hbm, ptab, idxbuf, g0, sem0):
    c = lax.axis_index("c")
    s = lax.axis_index("s")
    base = (c * 16 + s) * RPT
    @pl.when(s == 0)
    def _():
        pltpu.sync_copy(p_hbm, ptab)

    pltpu.sync_copy(idx_hbm.at[pl.ds(base, RPT)], idxbuf)
    plsc.subcore_barrier()

    def body(jo, carry):
        cps = []
        for u in range(5):
            j = jo * 5 + u
            cps.append(pltpu.async_copy(ptab.at[idxbuf.at[j]], g0.at[j],
                                        sem0))
        for cp in cps:
            cp.wait()
        return carry

    lax.fori_loop(0, RPT // 5, body, 0)
    pltpu.sync_copy(g0, d_hbm.at[pl.ds(base, RPT)])


CH_E = 64
RPT_E = EPT // CH_E
NBUF = 2


@functools.partial(
    pl.kernel,
    out_type=(
        jax.ShapeDtypeStruct((NPAD, D), jnp.float32),
        jax.ShapeDtypeStruct((NPAD, D), jnp.float32),
    ),
    mesh=_mesh,
    scratch_types=(
        [
            pltpu.VMEM_SHARED((NPAD, D), jnp.float32),
            pltpu.VMEM((RPT_E, CH_E), jnp.int32),
        ]
        + [pltpu.VMEM((CH_E, D), jnp.float32)] * NBUF
        + [pltpu.SemaphoreType.DMA] * (2 * NBUF)
    ),
)
def _seg_agg(wv_hbm, idx_hbm, zeros_hbm, pa0_hbm, pa1_hbm, acc, idxbuf,
             *bufs_sems):
    bufs = bufs_sems[:NBUF]
    lsem = bufs_sems[NBUF:2 * NBUF]
    ssem = bufs_sems[2 * NBUF:]
    c = lax.axis_index("c")
    s = lax.axis_index("s")
    tile = c * 16 + s
    ebase = tile * EPT

    pltpu.sync_copy(idx_hbm.at[pl.ds(tile * RPT_E, RPT_E)], idxbuf)
    pltpu.sync_copy(zeros_hbm.at[pl.ds(s * NPT, NPT)],
                    acc.at[pl.ds(s * NPT, NPT)])
    plsc.subcore_barrier()

    def src(j):
        return wv_hbm.at[pl.ds(ebase + j * CH_E, CH_E)]

    for u in range(NBUF):
        pltpu.async_copy(src(u), bufs[u], lsem[u])

    def body(g, carry):
        for u in range(NBUF):
            j = g * NBUF + u
            pltpu.make_async_copy(src(j), bufs[u], lsem[u]).wait()
            pltpu.async_copy(bufs[u], acc.at[idxbuf.at[j]], ssem[u],
                             add=True)
        for u in range(NBUF):
            j = g * NBUF + u

            @pl.when(j + NBUF < RPT_E)
            def _():
                pltpu.make_async_copy(bufs[u], acc.at[idxbuf.at[j]],
                                      ssem[u]).wait()
                pltpu.async_copy(src(j + NBUF), bufs[u], lsem[u])

        return carry

    lax.fori_loop(0, RPT_E // NBUF, body, 0)
    for u in range(NBUF):
        j = RPT_E - NBUF + u
        pltpu.make_async_copy(bufs[u], acc.at[idxbuf.at[j]], ssem[u]).wait()

    plsc.subcore_barrier()

    @pl.when(c == 0)
    def _():
        pltpu.sync_copy(acc.at[pl.ds(s * NPT, NPT)],
                        pa0_hbm.at[pl.ds(s * NPT, NPT)])

    @pl.when(c == 1)
    def _():
        pltpu.sync_copy(acc.at[pl.ds(s * NPT, NPT)],
                        pa1_hbm.at[pl.ds(s * NPT, NPT)])


def _attn_body(o8_ref, d_ref, attn_ref):
    ex_t = o8_ref[pl.ds(0, H), :]
    attn_ref[...] = ex_t / d_ref[...]


def _attn_norm(o8, d_row):
    beg = 12800
    return pl.pallas_call(
        _attn_body,
        grid=(E // beg,),
        in_specs=[
            pl.BlockSpec((8, beg), lambda i: (0, i)),
            pl.BlockSpec((1, beg), lambda i: (0, i)),
        ],
        out_specs=pl.BlockSpec((H, beg), lambda i: (0, i)),
        out_shape=jax.ShapeDtypeStruct((H, E), jnp.float32),
    )(o8, d_row)


def _combine_body(p0_ref, p1_ref, den_ref, out_ref):
    den = den_ref[...]
    inv = jnp.where(den > 0.0, 1.0 / den, 0.0)
    out_ref[...] = (p0_ref[...] + p1_ref[...]) * inv


def _combine(p0_pad, p1_pad, den_pad):
    bn = 10000
    return pl.pallas_call(
        _combine_body,
        grid=(N // bn,),
        in_specs=[
            pl.BlockSpec((bn, D), lambda i: (i, 0)),
            pl.BlockSpec((bn, D), lambda i: (i, 0)),
            pl.BlockSpec((bn, 1), lambda i: (i, 0)),
        ],
        out_specs=pl.BlockSpec((bn, D), lambda i: (i, 0)),
        out_shape=jax.ShapeDtypeStruct((N, D), jnp.float32),
    )(p0_pad, p1_pad, den_pad)


def kernel(h_src, Q_dst, Wk, Wv, W1, b1, W2, b2, src_idx, dst_idx,
           num_dst_nodes):
    del src_idx, num_dst_nodes
    q = Q_dst.reshape(E, D)
    w1h = W1[:, :D]
    w1q = W1[:, D:]
    rep = jnp.asarray(np.repeat(np.eye(H, dtype=np.float32), DK, axis=1))
    b1r = b1.reshape(1, D)
    b2r = b2.reshape(1, 1)

    o8, wv_rows = _edge_proj(h_src, q, Wk, Wv, w1h, w1q, W2, b1r, b2r, rep)

    pad_idx = N + (jnp.arange(E_PAD - E, dtype=jnp.int32) % (NPAD - N))
    idx_flat = jnp.concatenate([dst_idx.astype(jnp.int32), pad_idx])
    idx2d = idx_flat.reshape(NROWS, CHUNK)

    zeros_n = jnp.zeros((NPAD,), jnp.float32)
    p = _seg_sum(o8[H].reshape(NROWS, CHUNK), idx2d, zeros_n)

    d = _seg_gather(idx2d, p)
    attn_t = _attn_norm(o8, d.reshape(1, E_PAD))
    attn_norm = attn_t.T

    zeros_nd = jnp.zeros((NPAD, D), jnp.float32)
    idx2d_e = idx_flat.reshape(E_PAD // CH_E, CH_E)
    pa0, pa1 = _seg_agg(wv_rows, idx2d_e, zeros_nd)

    aggregated = _combine(pa0, pa1, p.reshape(NPAD, 1))
    return (aggregated, attn_norm, o8[H + 1, :E])


# speedup vs baseline: 1.0302x; 1.0302x over previous
import functools

import jax
import jax.numpy as jnp
import numpy as np
from jax import lax
from jax.experimental import pallas as pl
from jax.experimental.pallas import tpu as pltpu
from jax.experimental.pallas import tpu_sc as plsc

E = 320000
N = 10000
D = 128
H = 4
DK = 32

E_PAD = 327680
NPAD = 10112
CHUNK = 128
NROWS = E_PAD // CHUNK
NTILES = 32
RPT = NROWS // NTILES
RPT_B = NROWS // 16
EPT = E_PAD // NTILES
NPT = NPAD // 16

BE = 2560
NBLK = E // BE
NBLK_D = E_PAD // BE

_INV_SQRT_DK = 1.0 / np.sqrt(DK)
_CLAMP = 60.0

_mesh = plsc.VectorSubcoreMesh(core_axis_name="c", subcore_axis_name="s")


def _edge_proj_body(h_ref, q_ref, wk_ref, wv_ref, w1h_ref, w1q_ref, w2_ref,
                    b1_ref, b2_ref, rep_ref, o8_ref, out_ref):
    h = h_ref[...]
    q = q_ref[...]
    k = lax.dot_general(h, wk_ref[...], (((1,), (1,)), ((), ())),
                        preferred_element_type=jnp.float32)
    prod = q * k
    s_t = lax.dot_general(rep_ref[...], prod, (((1,), (1,)), ((), ())),
                          preferred_element_type=jnp.float32) * _INV_SQRT_DK
    ex_t = jnp.exp(jnp.minimum(s_t, _CLAMP))
    o8_ref[pl.ds(0, H), :] = ex_t
    o8_ref[pl.ds(H, 1), :] = jnp.sum(ex_t, axis=0, keepdims=True)
    hid = lax.dot_general(h, w1h_ref[...], (((1,), (1,)), ((), ())),
                          preferred_element_type=jnp.float32)
    hid = hid + lax.dot_general(q, w1q_ref[...], (((1,), (1,)), ((), ())),
                                preferred_element_type=jnp.float32)
    hid = jnp.maximum(hid + b1_ref[...], 0.0)
    wp_t = lax.dot_general(w2_ref[...], hid, (((1,), (1,)), ((), ())),
                           preferred_element_type=jnp.float32)
    o8_ref[pl.ds(H + 1, 1), :] = wp_t + b2_ref[0, 0]
    v = lax.dot_general(h, wv_ref[...], (((1,), (1,)), ((), ())),
                        preferred_element_type=jnp.float32)
    scale = lax.dot_general(ex_t, rep_ref[...], (((0,), (0,)), ((), ())),
                            preferred_element_type=jnp.float32)
    out_ref[...] = v * scale


def _edge_proj(h, q, wk, wv, w1h, w1q, w2, b1, b2, rep):
    full = lambda shp: pl.BlockSpec(shp, lambda i: (0, 0))
    clamped = lambda i: (jnp.minimum(i, NBLK - 1), 0)
    return pl.pallas_call(
        _edge_proj_body,
        grid=(NBLK_D,),
        in_specs=[
            pl.BlockSpec((BE, D), clamped),
            pl.BlockSpec((BE, D), clamped),
            full((D, D)), full((D, D)), full((D, D)), full((D, D)),
            full((1, D)), full((1, D)), full((1, 1)),
            full((H, D)),
        ],
        out_specs=[
            pl.BlockSpec((8, BE), lambda i: (0, i)),
            pl.BlockSpec((BE, D), lambda i: (i, 0)),
        ],
        out_shape=[
            jax.ShapeDtypeStruct((8, E_PAD), jnp.float32),
            jax.ShapeDtypeStruct((E_PAD, D), jnp.float32),
        ],
    )(h, q, wk, wv, w1h, w1q, w2, b1, b2, rep)


@functools.partial(
    pl.kernel,
    out_type=jax.ShapeDtypeStruct((NPAD,), jnp.float32),
    mesh=_mesh,
    scratch_types=[
        pltpu.VMEM_SHARED((NPAD,), jnp.float32),
        pltpu.VMEM((RPT_B, CHUNK), jnp.int32),
        pltpu.VMEM((RPT_B, CHUNK), jnp.float32),
    ],
)
def _seg_sum(sexp_hbm, idx_hbm, zeros_hbm, p_hbm, acc, idxbuf, updbuf):
    c = lax.axis_index("c")
    s = lax.axis_index("s")

    @pl.when(c == 0)
    def _():
        base = s * RPT_B

        @pl.when(s == 0)
        def _():
            pltpu.sync_copy(zeros_hbm, acc)

        pltpu.sync_copy(idx_hbm.at[pl.ds(base, RPT_B)], idxbuf)
        pltpu.sync_copy(sexp_hbm.at[pl.ds(base, RPT_B)], updbuf)
        plsc.subcore_barrier()

        def body(j, carry):
            pltpu.sync_copy(updbuf.at[j], acc.at[idxbuf.at[j]], add=True)
            return carry

        lax.fori_loop(0, RPT_B, body, 0)
        plsc.subcore_barrier()

        @pl.when(s == 0)
        def _():
            pltpu.sync_copy(acc, p_hbm)


@functools.partial(
    pl.kernel,
    out_type=jax.ShapeDtypeStruct((NROWS, CHUNK), jnp.float32),
    mesh=_mesh,
    scratch_types=[
        pltpu.VMEM_SHARED((NPAD,), jnp.float32),
        pltpu.VMEM((RPT, CHUNK), jnp.int32),
        pltpu.VMEM((RPT, CHUNK), jnp.float32),
        pltpu.SemaphoreType.DMA,
    ],
)
def _seg_gather(idx_hbm, p_hbm, d_hbm, ptab, idxbuf, g0, sem0):
    c = lax.axis_index("c")
    s = lax.axis_index("s")
    base = (c * 16 + s) * RPT
    @pl.when(s == 0)
    def _():
        pltpu.sync_copy(p_hbm, ptab)

    pltpu.sync_copy(idx_hbm.at[pl.ds(base, RPT)], idxbuf)
    plsc.subcore_barrier()

    def body(jo, carry):
        cps = []
        for u in range(5):
            j = jo * 5 + u
            cps.append(pltpu.async_copy(ptab.at[idxbuf.at[j]], g0.at[j],
                                        sem0))
        for cp in cps:
            cp.wait()
        return carry

    lax.fori_loop(0, RPT // 5, body, 0)
    pltpu.sync_copy(g0, d_hbm.at[pl.ds(base, RPT)])


@functools.partial(
    pl.kernel,
    out_type=(
        jax.ShapeDtypeStruct((NPAD, D), jnp.float32),
        jax.ShapeDtypeStruct((NPAD, D), jnp.float32),
    ),
    mesh=_mesh,
    scratch_types=[
        pltpu.VMEM_SHARED((NPAD, D), jnp.float32),
        pltpu.VMEM((RPT, CHUNK), jnp.int32),
        pltpu.VMEM((CHUNK, D), jnp.float32),
        pltpu.VMEM((CHUNK, D), jnp.float32),
        pltpu.SemaphoreType.DMA,
        pltpu.SemaphoreType.DMA,
        pltpu.SemaphoreType.DMA,
        pltpu.SemaphoreType.DMA,
    ],
)
def _seg_agg(wv_hbm, idx_hbm, zeros_hbm, pa0_hbm, pa1_hbm, acc, idxbuf,
             buf0, buf1, sem0, sem1, ssem0, ssem1):
    c = lax.axis_index("c")
    s = lax.axis_index("s")
    tile = c * 16 + s
    ebase = tile * EPT

    pltpu.sync_copy(idx_hbm.at[pl.ds(tile * RPT, RPT)], idxbuf)
    pltpu.sync_copy(zeros_hbm.at[pl.ds(s * NPT, NPT)],
                    acc.at[pl.ds(s * NPT, NPT)])
    plsc.subcore_barrier()

    def src(j):
        return wv_hbm.at[pl.ds(ebase + j * CHUNK, CHUNK)]

    pltpu.async_copy(src(0), buf0, sem0)
    pltpu.async_copy(src(1), buf1, sem1)

    def body(i2, carry):
        j0 = i2 * 2
        j1 = j0 + 1
        pltpu.make_async_copy(src(j0), buf0, sem0).wait()
        cp0 = pltpu.async_copy(buf0, acc.at[idxbuf.at[j0]], ssem0, add=True)
        pltpu.make_async_copy(src(j1), buf1, sem1).wait()
        cp1 = pltpu.async_copy(buf1, acc.at[idxbuf.at[j1]], ssem1, add=True)
        cp0.wait()

        @pl.when(j0 + 2 < RPT)
        def _():
            pltpu.async_copy(src(j0 + 2), buf0, sem0)

        cp1.wait()

        @pl.when(j1 + 2 < RPT)
        def _():
            pltpu.async_copy(src(j1 + 2), buf1, sem1)

        return carry

    lax.fori_loop(0, RPT // 2, body, 0)
    plsc.subcore_barrier()

    @pl.when(c == 0)
    def _():
        pltpu.sync_copy(acc.at[pl.ds(s * NPT, NPT)],
                        pa0_hbm.at[pl.ds(s * NPT, NPT)])

    @pl.when(c == 1)
    def _():
        pltpu.sync_copy(acc.at[pl.ds(s * NPT, NPT)],
                        pa1_hbm.at[pl.ds(s * NPT, NPT)])


def _attn_body(o8_ref, d_ref, attn_ref):
    ex_t = o8_ref[pl.ds(0, H), :]
    attn_ref[...] = ex_t / d_ref[...]


def _attn_norm(o8, d_row):
    beg = 12800
    return pl.pallas_call(
        _attn_body,
        grid=(E // beg,),
        in_specs=[
            pl.BlockSpec((8, beg), lambda i: (0, i)),
            pl.BlockSpec((1, beg), lambda i: (0, i)),
        ],
        out_specs=pl.BlockSpec((H, beg), lambda i: (0, i)),
        out_shape=jax.ShapeDtypeStruct((H, E), jnp.float32),
    )(o8, d_row)


def _combine_body(p0_ref, p1_ref, den_ref, out_ref):
    den = den_ref[...]
    inv = jnp.where(den > 0.0, 1.0 / den, 0.0)
    out_ref[...] = (p0_ref[...] + p1_ref[...]) * inv


def _combine(p0_pad, p1_pad, den_pad):
    bn = 10000
    return pl.pallas_call(
        _combine_body,
        grid=(N // bn,),
        in_specs=[
            pl.BlockSpec((bn, D), lambda i: (i, 0)),
            pl.BlockSpec((bn, D), lambda i: (i, 0)),
            pl.BlockSpec((bn, 1), lambda i: (i, 0)),
        ],
        out_specs=pl.BlockSpec((bn, D), lambda i: (i, 0)),
        out_shape=jax.ShapeDtypeStruct((N, D), jnp.float32),
    )(p0_pad, p1_pad, den_pad)


def kernel(h_src, Q_dst, Wk, Wv, W1, b1, W2, b2, src_idx, dst_idx,
           num_dst_nodes):
    del src_idx, num_dst_nodes
    q = Q_dst.reshape(E, D)
    w1h = W1[:, :D]
    w1q = W1[:, D:]
    rep = jnp.asarray(np.repeat(np.eye(H, dtype=np.float32), DK, axis=1))
    b1r = b1.reshape(1, D)
    b2r = b2.reshape(1, 1)

    o8, wv_rows = _edge_proj(h_src, q, Wk, Wv, w1h, w1q, W2, b1r, b2r, rep)

    pad_idx = N + (jnp.arange(E_PAD - E, dtype=jnp.int32) % (NPAD - N))
    idx_flat = jnp.concatenate([dst_idx.astype(jnp.int32), pad_idx])
    idx2d = idx_flat.reshape(NROWS, CHUNK)

    zeros_n = jnp.zeros((NPAD,), jnp.float32)
    p = _seg_sum(o8[H].reshape(NROWS, CHUNK), idx2d, zeros_n)

    d = _seg_gather(idx2d, p)
    attn_t = _attn_norm(o8, d.reshape(1, E_PAD))
    attn_norm = attn_t.T

    zeros_nd = jnp.zeros((NPAD, D), jnp.float32)
    pa0, pa1 = _seg_agg(wv_rows, idx2d, zeros_nd)

    aggregated = _combine(pa0, pa1, p.reshape(NPAD, 1))
    return (aggregated, attn_norm, o8[H + 1, :E])
